# baseline (speedup 1.0000x reference)
import functools

import jax
import jax.numpy as jnp
from jax import lax
from jax.experimental import pallas as pl
from jax.experimental.pallas import tpu as pltpu
from jax.experimental.pallas import tpu_sc as plsc

N = 10000
D = 128
H = 128
G = 64
S = 32
C = 2
E = 320000

NC = 2
NS = 16
NW = NC * NS

PT = 10368
EP = PT * NW
K = 96
NCHUNK = PT // K
NCHUNK0 = 150
NCHUNK1 = 2 * NCHUNK - NCHUNK0
PT0 = NCHUNK0 * K
PT1 = NCHUNK1 * K
NB = 3
PF = 2
RS = 6
PFI = 4
KD = 128
NCHUNKD = PT // KD
NBD = 3

NPAD = 10240
ZR = NPAD // NS

_mesh = plsc.VectorSubcoreMesh(
    core_axis_name="c", subcore_axis_name="s", num_cores=NC, num_subcores=NS)


@functools.partial(
    pl.kernel,
    out_type=jax.ShapeDtypeStruct((NC, NPAD, 16), jnp.float32),
    mesh=_mesh,
    scratch_types=[
        pltpu.VMEM((NCHUNKD, KD), jnp.int32),
        pltpu.VMEM((KD, 16), jnp.float32),
        [pltpu.SemaphoreType.DMA] * NBD,
        pltpu.VMEM_SHARED((NPAD, 16), jnp.float32),
    ],
)
def _sc_degree(dst_hbm, zeros16_hbm, ones16_hbm, out_hbm, idx_v, ones_v,
               sems, acc_sh):
    c = lax.axis_index("c")
    s = lax.axis_index("s")
    pltpu.sync_copy(zeros16_hbm, acc_sh.at[pl.ds(s * ZR, ZR)])
    pltpu.sync_copy(ones16_hbm, ones_v)
    pltpu.sync_copy(dst_hbm.at[c * NS + s], idx_v)
    plsc.subcore_barrier()

    def group(g, _):
        for b in range(NBD):
            j = g * NBD + b
            @pl.when(g > 0)
            def _wait():
                pltpu.make_async_copy(ones_v, acc_sh.at[idx_v.at[0]],
                                      sems[b]).wait()
            pltpu.async_copy(ones_v, acc_sh.at[idx_v.at[j]], sems[b], add=True)
        return _

    lax.fori_loop(0, NCHUNKD // NBD, group, None)
    for b in range(NBD):
        pltpu.make_async_copy(ones_v, acc_sh.at[idx_v.at[0]], sems[b]).wait()
    plsc.subcore_barrier()
    pltpu.sync_copy(acc_sh.at[pl.ds(s * ZR, ZR)], out_hbm.at[c, pl.ds(s * ZR, ZR)])


@functools.partial(
    pl.kernel,
    out_type=jax.ShapeDtypeStruct((NC, NPAD, H), jnp.float32),
    mesh=_mesh,
    scratch_types=[
        [pltpu.VMEM((K,), jnp.int32)] * RS,
        [pltpu.VMEM((K,), jnp.int32)] * RS,
        [pltpu.VMEM((K, H), jnp.float32)] * NB,
        [pltpu.SemaphoreType.DMA] * RS,
        [pltpu.SemaphoreType.DMA] * NB,
        [pltpu.SemaphoreType.DMA] * NB,
        pltpu.VMEM_SHARED((NPAD, H), jnp.float32),
    ],
)
def _sc_scatter(hs_hbm, idx_hbm, zeros_hbm, out_hbm,
                sstage, dstage, rows, isem, gsem, ssem, acc_sh):
    c = lax.axis_index("c")
    s = lax.axis_index("s")
    wid = c * NS + s
    ptc = jnp.where(c == 0, PT0, PT1)
    ngroups = jnp.where(c == 0, NCHUNK0 // RS, NCHUNK1 // RS)
    nchunk = ngroups * RS
    pltpu.sync_copy(zeros_hbm, acc_sh.at[pl.ds(s * ZR, ZR)])
    plsc.subcore_barrier()

    ibase = wid * (2 * PT0)

    def fire_idx(ji, r):
        pltpu.async_copy(idx_hbm.at[pl.ds(ibase + K * ji, K)], sstage[r],
                         isem[r])
        pltpu.async_copy(idx_hbm.at[pl.ds(ibase + ptc + K * ji, K)],
                         dstage[r], isem[r])

    def wait_idx(r):
        pltpu.make_async_copy(idx_hbm.at[pl.ds(0, K)], sstage[r],
                              isem[r]).wait()
        pltpu.make_async_copy(idx_hbm.at[pl.ds(0, K)], dstage[r],
                              isem[r]).wait()

    for r in range(PFI):
        fire_idx(r, r)
    for b in range(PF):
        wait_idx(b)
        pltpu.async_copy(hs_hbm.at[sstage[b]], rows[b], gsem[b])

    def group(g, _):
        for u in range(RS):
            j = g * RS + u
            b = u % NB
            r = u
            rp = (u + PF) % RS
            bp = (b + PF) % NB
            ri = (u + PFI) % RS
            @pl.when(j + PFI < nchunk)
            def _pf_idx():
                fire_idx(j + PFI, ri)
            @pl.when(j + PF < nchunk)
            def _pf_gather():
                @pl.when(j + PF >= NB)
                def _wait_scatter():
                    pltpu.make_async_copy(rows[bp], acc_sh.at[dstage[rp]],
                                          ssem[bp]).wait()
                wait_idx(rp)
                pltpu.async_copy(hs_hbm.at[sstage[rp]], rows[bp], gsem[bp])
            pltpu.make_async_copy(hs_hbm.at[sstage[r]], rows[b],
                                  gsem[b]).wait()
            pltpu.async_copy(rows[b], acc_sh.at[dstage[r]], ssem[b], add=True)
        return _

    lax.fori_loop(0, ngroups, group, None)
    for b in range(NB):
        pltpu.make_async_copy(rows[b], acc_sh.at[dstage[b]], ssem[b]).wait()
    plsc.subcore_barrier()
    pltpu.sync_copy(acc_sh.at[pl.ds(s * ZR, ZR)], out_hbm.at[c, pl.ds(s * ZR, ZR)])


def _tc_pre_body(d0_ref, d1_ref, x_ref, w1_ref, stat_ref, ws_ref, bs_ref,
                 dinv_ref, hs1_ref, statout_ref):
    deg = d0_ref[:N, 0:1] + d1_ref[:N, 0:1]
    dinv = lax.rsqrt(deg)
    dinv_ref[...] = dinv
    h = jnp.dot(x_ref[...], w1_ref[...], preferred_element_type=jnp.float32)
    hs1_ref[...] = h * dinv
    st = jnp.dot(stat_ref[...], ws_ref[...], preferred_element_type=jnp.float32)
    statout_ref[...] = jnp.maximum(st + bs_ref[...], 0.0)


def _tc_mid_body(a0_ref, a1_ref, dinv_ref, b_ref, wn_ref, batch_ref,
                 hsn_ref, pool_ref):
    dinv = dinv_ref[...]
    xl = dinv * (a0_ref[:N, :] + a1_ref[:N, :]) + b_ref[...]
    xr = jnp.maximum(xl, 0.0)
    gid = lax.broadcasted_iota(jnp.int32, (N, G), 1)
    onehot = jnp.where(batch_ref[...] == gid, 1.0, 0.0)
    pool_ref[...] = lax.dot_general(
        onehot, xr, (((0,), (0,)), ((), ())),
        preferred_element_type=jnp.float32)
    hsn_ref[...] = jnp.dot(
        xl, wn_ref[...], preferred_element_type=jnp.float32) * dinv


def _tc_final_body(a0_ref, a1_ref, dinv_ref, b3_ref, batch_ref, p1_ref,
                   p2_ref, stat_ref, wc_ref, bc_ref, out_ref):
    dinv = dinv_ref[...]
    x3 = dinv * (a0_ref[:N, :] + a1_ref[:N, :]) + b3_ref[...]
    x3r = jnp.maximum(x3, 0.0)
    gid = lax.broadcasted_iota(jnp.int32, (N, G), 1)
    onehot = jnp.where(batch_ref[...] == gid, 1.0, 0.0)
    pool3 = lax.dot_general(onehot, x3r, (((0,), (0,)), ((), ())),
                            preferred_element_type=jnp.float32)
    cnt = lax.dot_general(onehot, jnp.ones((N, 1), jnp.float32),
                          (((0,), (0,)), ((), ())),
                          preferred_element_type=jnp.float32)
    rc = 1.0 / jnp.maximum(cnt, 1.0)
    comb = jnp.concatenate(
        [p1_ref[...] * rc, p2_ref[...] * rc, pool3 * rc, stat_ref[...]], axis=1)
    out_ref[...] = jnp.dot(
        comb, wc_ref[...], preferred_element_type=jnp.float32) + bc_ref[...]


_f32 = jnp.float32

_tc_pre = pl.pallas_call(
    _tc_pre_body,
    out_shape=[
        jax.ShapeDtypeStruct((N, 1), _f32),
        jax.ShapeDtypeStruct((N, H), _f32),
        jax.ShapeDtypeStruct((G, H), _f32),
    ],
)

_tc_mid = pl.pallas_call(
    _tc_mid_body,
    out_shape=[
        jax.ShapeDtypeStruct((N, H), _f32),
        jax.ShapeDtypeStruct((G, H), _f32),
    ],
)

_tc_final = pl.pallas_call(
    _tc_final_body,
    out_shape=jax.ShapeDtypeStruct((G, C), _f32),
)


def kernel(x, edge_index, statistical, batch, W1, b1, W2, b2, W3, b3,
           Ws, bs, Wc, bc):
    ei = edge_index.astype(jnp.int32)
    si = jnp.arange(N, dtype=jnp.int32)
    npadjunk = EP - E - N
    srcf = jnp.concatenate([ei[0], si, jnp.zeros((npadjunk,), jnp.int32)])
    junk = N + jnp.arange(npadjunk, dtype=jnp.int32) % (NPAD - N)
    dstf = jnp.concatenate([ei[1], si, junk])
    e0 = NS * PT0
    s0 = srcf[:e0].reshape(NS, PT0)
    d0 = dstf[:e0].reshape(NS, PT0)
    s1 = srcf[e0:].reshape(NS, PT1)
    d1 = dstf[e0:].reshape(NS, PT1)
    pad1 = jnp.zeros((NS, PT0 - PT1), jnp.int32)
    idxp = jnp.concatenate([
        jnp.concatenate([s0, d0], axis=1),
        jnp.concatenate([s1, d1, pad1, pad1], axis=1)], axis=0).reshape(-1)
    dstd = dstf.reshape(NW, NCHUNKD, KD)

    zeros16 = jnp.zeros((ZR, 16), _f32)
    ones16 = jnp.ones((KD, 16), _f32)
    zrows = jnp.zeros((ZR, H), _f32)
    batch2d = batch.astype(jnp.int32).reshape(N, 1)

    degp = _sc_degree(dstd, zeros16, ones16)
    dinv, hs1, stat = _tc_pre(degp[0], degp[1], x, W1, statistical, Ws,
                              bs.reshape(1, H))

    acc1 = _sc_scatter(hs1, idxp, zrows)
    hs2, pool1 = _tc_mid(acc1[0], acc1[1], dinv, b1.reshape(1, H), W2, batch2d)

    acc2 = _sc_scatter(hs2, idxp, zrows)
    hs3, pool2 = _tc_mid(acc2[0], acc2[1], dinv, b2.reshape(1, H), W3, batch2d)

    acc3 = _sc_scatter(hs3, idxp, zrows)
    out = _tc_final(acc3[0], acc3[1], dinv, b3.reshape(1, H), batch2d,
                    pool1, pool2, stat, Wc, bc.reshape(1, C))
    return out


# speedup vs baseline: 1.0374x; 1.0374x over previous
import functools

import jax
import jax.numpy as jnp
from jax import lax
from jax.experimental import pallas as pl
from jax.experimental.pallas import tpu as pltpu
from jax.experimental.pallas import tpu_sc as plsc

N = 10000
D = 128
H = 128
G = 64
S = 32
C = 2
E = 320000

NC = 2
NS = 16
NW = NC * NS

PT = 10368
EP = PT * NW
K = 96
NCHUNK = PT // K
NCHUNK0 = 144
NCHUNK1 = 2 * NCHUNK - NCHUNK0
PT0 = NCHUNK0 * K
PT1 = NCHUNK1 * K
NB = 3
PF = 2
RS = 6
PFI = 4
KD = 128
NCHUNKD = PT // KD
NBD = 3

NPAD = 10240
ZR = NPAD // NS

_mesh = plsc.VectorSubcoreMesh(
    core_axis_name="c", subcore_axis_name="s", num_cores=NC, num_subcores=NS)


@functools.partial(
    pl.kernel,
    out_type=jax.ShapeDtypeStruct((NC, NPAD, 16), jnp.float32),
    mesh=_mesh,
    scratch_types=[
        pltpu.VMEM((NCHUNKD, KD), jnp.int32),
        pltpu.VMEM((KD, 16), jnp.float32),
        [pltpu.SemaphoreType.DMA] * NBD,
        pltpu.VMEM_SHARED((NPAD, 16), jnp.float32),
    ],
)
def _sc_degree(dst_hbm, zeros16_hbm, ones16_hbm, out_hbm, idx_v, ones_v,
               sems, acc_sh):
    c = lax.axis_index("c")
    s = lax.axis_index("s")
    pltpu.sync_copy(zeros16_hbm, acc_sh.at[pl.ds(s * ZR, ZR)])
    pltpu.sync_copy(ones16_hbm, ones_v)
    pltpu.sync_copy(dst_hbm.at[c * NS + s], idx_v)
    plsc.subcore_barrier()

    def group(g, _):
        for b in range(NBD):
            j = g * NBD + b
            @pl.when(g > 0)
            def _wait():
                pltpu.make_async_copy(ones_v, acc_sh.at[idx_v.at[0]],
                                      sems[b]).wait()
            pltpu.async_copy(ones_v, acc_sh.at[idx_v.at[j]], sems[b], add=True)
        return _

    lax.fori_loop(0, NCHUNKD // NBD, group, None)
    for b in range(NBD):
        pltpu.make_async_copy(ones_v, acc_sh.at[idx_v.at[0]], sems[b]).wait()
    plsc.subcore_barrier()
    pltpu.sync_copy(acc_sh.at[pl.ds(s * ZR, ZR)], out_hbm.at[c, pl.ds(s * ZR, ZR)])


@functools.partial(
    pl.kernel,
    out_type=jax.ShapeDtypeStruct((NC, NPAD, H), jnp.float32),
    mesh=_mesh,
    scratch_types=[
        [pltpu.VMEM((K,), jnp.int32)] * RS,
        [pltpu.VMEM((K,), jnp.int32)] * RS,
        [pltpu.VMEM((K, H), jnp.float32)] * NB,
        [pltpu.SemaphoreType.DMA] * RS,
        [pltpu.SemaphoreType.DMA] * NB,
        [pltpu.SemaphoreType.DMA] * NB,
        pltpu.VMEM_SHARED((NPAD, H), jnp.float32),
    ],
)
def _sc_scatter(hs_hbm, idx_hbm, zeros_hbm, out_hbm,
                sstage, dstage, rows, isem, gsem, ssem, acc_sh):
    c = lax.axis_index("c")
    s = lax.axis_index("s")
    wid = c * NS + s
    ptc = jnp.where(c == 0, PT0, PT1)
    ngroups = jnp.where(c == 0, NCHUNK0 // RS, NCHUNK1 // RS)
    nchunk = ngroups * RS
    pltpu.sync_copy(zeros_hbm, acc_sh.at[pl.ds(s * ZR, ZR)])
    plsc.subcore_barrier()

    ibase = wid * (2 * PT0)

    def fire_idx(ji, r):
        pltpu.async_copy(idx_hbm.at[pl.ds(ibase + K * ji, K)], sstage[r],
                         isem[r])
        pltpu.async_copy(idx_hbm.at[pl.ds(ibase + ptc + K * ji, K)],
                         dstage[r], isem[r])

    def wait_idx(r):
        pltpu.make_async_copy(idx_hbm.at[pl.ds(0, K)], sstage[r],
                              isem[r]).wait()
        pltpu.make_async_copy(idx_hbm.at[pl.ds(0, K)], dstage[r],
                              isem[r]).wait()

    for r in range(PFI):
        fire_idx(r, r)
    for b in range(PF):
        wait_idx(b)
        pltpu.async_copy(hs_hbm.at[sstage[b]], rows[b], gsem[b])

    def group(g, _):
        for u in range(RS):
            j = g * RS + u
            b = u % NB
            r = u
            rp = (u + PF) % RS
            bp = (b + PF) % NB
            ri = (u + PFI) % RS
            @pl.when(j + PFI < nchunk)
            def _pf_idx():
                fire_idx(j + PFI, ri)
            @pl.when(j + PF < nchunk)
            def _pf_gather():
                @pl.when(j + PF >= NB)
                def _wait_scatter():
                    pltpu.make_async_copy(rows[bp], acc_sh.at[dstage[rp]],
                                          ssem[bp]).wait()
                wait_idx(rp)
                pltpu.async_copy(hs_hbm.at[sstage[rp]], rows[bp], gsem[bp])
            pltpu.make_async_copy(hs_hbm.at[sstage[r]], rows[b],
                                  gsem[b]).wait()
            pltpu.async_copy(rows[b], acc_sh.at[dstage[r]], ssem[b], add=True)
        return _

    lax.fori_loop(0, ngroups, group, None)
    for b in range(NB):
        pltpu.make_async_copy(rows[b], acc_sh.at[dstage[b]], ssem[b]).wait()
    plsc.subcore_barrier()
    pltpu.sync_copy(acc_sh.at[pl.ds(s * ZR, ZR)], out_hbm.at[c, pl.ds(s * ZR, ZR)])


def _tc_pre_body(d0_ref, d1_ref, x_ref, w1_ref, stat_ref, ws_ref, bs_ref,
                 dinv_ref, hs1_ref, statout_ref):
    deg = d0_ref[:N, 0:1] + d1_ref[:N, 0:1]
    dinv = lax.rsqrt(deg)
    dinv_ref[...] = dinv
    h = jnp.dot(x_ref[...], w1_ref[...], preferred_element_type=jnp.float32)
    hs1_ref[...] = h * dinv
    st = jnp.dot(stat_ref[...], ws_ref[...], preferred_element_type=jnp.float32)
    statout_ref[...] = jnp.maximum(st + bs_ref[...], 0.0)


def _tc_mid_body(a0_ref, a1_ref, dinv_ref, b_ref, wn_ref, batch_ref,
                 hsn_ref, pool_ref):
    dinv = dinv_ref[...]
    xl = dinv * (a0_ref[:N, :] + a1_ref[:N, :]) + b_ref[...]
    xr = jnp.maximum(xl, 0.0)
    gid = lax.broadcasted_iota(jnp.int32, (N, G), 1)
    onehot = jnp.where(batch_ref[...] == gid, 1.0, 0.0)
    pool_ref[...] = lax.dot_general(
        onehot, xr, (((0,), (0,)), ((), ())),
        preferred_element_type=jnp.float32)
    hsn_ref[...] = jnp.dot(
        xl, wn_ref[...], preferred_element_type=jnp.float32) * dinv


def _tc_final_body(a0_ref, a1_ref, dinv_ref, b3_ref, batch_ref, p1_ref,
                   p2_ref, stat_ref, wc_ref, bc_ref, out_ref):
    dinv = dinv_ref[...]
    x3 = dinv * (a0_ref[:N, :] + a1_ref[:N, :]) + b3_ref[...]
    x3r = jnp.maximum(x3, 0.0)
    gid = lax.broadcasted_iota(jnp.int32, (N, G), 1)
    onehot = jnp.where(batch_ref[...] == gid, 1.0, 0.0)
    pool3 = lax.dot_general(onehot, x3r, (((0,), (0,)), ((), ())),
                            preferred_element_type=jnp.float32)
    cnt = lax.dot_general(onehot, jnp.ones((N, 1), jnp.float32),
                          (((0,), (0,)), ((), ())),
                          preferred_element_type=jnp.float32)
    rc = 1.0 / jnp.maximum(cnt, 1.0)
    comb = jnp.concatenate(
        [p1_ref[...] * rc, p2_ref[...] * rc, pool3 * rc, stat_ref[...]], axis=1)
    out_ref[...] = jnp.dot(
        comb, wc_ref[...], preferred_element_type=jnp.float32) + bc_ref[...]


_f32 = jnp.float32

_tc_pre = pl.pallas_call(
    _tc_pre_body,
    out_shape=[
        jax.ShapeDtypeStruct((N, 1), _f32),
        jax.ShapeDtypeStruct((N, H), _f32),
        jax.ShapeDtypeStruct((G, H), _f32),
    ],
)

_tc_mid = pl.pallas_call(
    _tc_mid_body,
    out_shape=[
        jax.ShapeDtypeStruct((N, H), _f32),
        jax.ShapeDtypeStruct((G, H), _f32),
    ],
)

_tc_final = pl.pallas_call(
    _tc_final_body,
    out_shape=jax.ShapeDtypeStruct((G, C), _f32),
)


def kernel(x, edge_index, statistical, batch, W1, b1, W2, b2, W3, b3,
           Ws, bs, Wc, bc):
    ei = edge_index.astype(jnp.int32)
    si = jnp.arange(N, dtype=jnp.int32)
    npadjunk = EP - E - N
    srcf = jnp.concatenate([ei[0], si, jnp.zeros((npadjunk,), jnp.int32)])
    junk = N + jnp.arange(npadjunk, dtype=jnp.int32) % (NPAD - N)
    dstf = jnp.concatenate([ei[1], si, junk])
    e0 = NS * PT0
    s0 = srcf[:e0].reshape(NS, PT0)
    d0 = dstf[:e0].reshape(NS, PT0)
    s1 = srcf[e0:].reshape(NS, PT1)
    d1 = dstf[e0:].reshape(NS, PT1)
    pad1 = jnp.zeros((NS, PT0 - PT1), jnp.int32)
    idxp = jnp.concatenate([
        jnp.concatenate([s0, d0], axis=1),
        jnp.concatenate([s1, d1, pad1, pad1], axis=1)], axis=0).reshape(-1)
    dstd = dstf.reshape(NW, NCHUNKD, KD)

    zeros16 = jnp.zeros((ZR, 16), _f32)
    ones16 = jnp.ones((KD, 16), _f32)
    zrows = jnp.zeros((ZR, H), _f32)
    batch2d = batch.astype(jnp.int32).reshape(N, 1)

    degp = _sc_degree(dstd, zeros16, ones16)
    dinv, hs1, stat = _tc_pre(degp[0], degp[1], x, W1, statistical, Ws,
                              bs.reshape(1, H))

    acc1 = _sc_scatter(hs1, idxp, zrows)
    hs2, pool1 = _tc_mid(acc1[0], acc1[1], dinv, b1.reshape(1, H), W2, batch2d)

    acc2 = _sc_scatter(hs2, idxp, zrows)
    hs3, pool2 = _tc_mid(acc2[0], acc2[1], dinv, b2.reshape(1, H), W3, batch2d)

    acc3 = _sc_scatter(hs3, idxp, zrows)
    out = _tc_final(acc3[0], acc3[1], dinv, b3.reshape(1, H), batch2d,
                    pool1, pool2, stat, Wc, bc.reshape(1, C))
    return out
